# Initial kernel scaffold; baseline (speedup 1.0000x reference)
import jax
import jax.numpy as jnp
from jax.experimental import pallas as pl


def kernel(x, train_pos_edge_index, pos_edge_index, neg_edge_index, W1, b1, W2, b2, W3, b3, Wc1, bc1, Wc2, bc2, W4, b4):
    raise NotImplementedError("write your pallas kernel here")


# speedup vs baseline: 9.2828x; 9.2828x over previous
import functools

import jax
import jax.numpy as jnp
from jax import lax
from jax.experimental import pallas as pl
from jax.experimental.pallas import tpu as pltpu
from jax.experimental.pallas import tpu_sc as plsc

N = 10000
E = 320000
E2 = 2 * E
DF = 128
D = 64

NROWS = 10240
RB = 1024
PADI = 10008

NC = 2
NS = 16
NW = NC * NS
BK = 128

T_AGG = 10112
E_AGG = NW * T_AGG
T_DEC = 20096
E_DEC = NW * T_DEC

_mesh = plsc.VectorSubcoreMesh(core_axis_name="c", subcore_axis_name="s")


def _worker_id():
    return lax.axis_index("s") * NC + lax.axis_index("c")


def _fill(ref, nwords, value, dtype):
    v = jnp.full((16,), value, dtype)
    for j in range(nwords // 16):
        ref[pl.ds(j * 16, 16)] = v


@functools.partial(
    pl.kernel,
    mesh=_mesh,
    out_type=jax.ShapeDtypeStruct((NC, NROWS), jnp.float32),
    scratch_types=[
        pltpu.VMEM((BK,), jnp.int32),
        pltpu.VMEM((BK,), jnp.float32),
        pltpu.VMEM_SHARED((NROWS,), jnp.float32),
    ],
)
def _deg_kernel(dst_hbm, out_hbm, idx_v, val_v, acc_sh):
    cid = lax.axis_index("c")
    sid = lax.axis_index("s")
    wid = sid * NC + cid
    _fill(val_v, BK, 0.0, jnp.float32)
    for r in range(NROWS // NS // BK):
        pltpu.sync_copy(val_v, acc_sh.at[pl.ds(sid * (NROWS // NS) + r * BK, BK)])
    plsc.subcore_barrier()
    _fill(val_v, BK, 1.0, jnp.float32)

    def body(j, carry):
        base = wid * T_AGG + j * BK
        pltpu.sync_copy(dst_hbm.at[pl.ds(base, BK)], idx_v)
        pltpu.sync_copy(val_v, acc_sh.at[idx_v], add=True)
        return carry

    lax.fori_loop(0, T_AGG // BK, body, 0)
    plsc.subcore_barrier()
    for r in range(NROWS // NS // BK):
        off = sid * (NROWS // NS) + r * BK
        pltpu.sync_copy(acc_sh.at[pl.ds(off, BK)], val_v)
        pltpu.sync_copy(val_v, out_hbm.at[cid, pl.ds(off, BK)])


@functools.partial(
    pl.kernel,
    mesh=_mesh,
    compiler_params=pltpu.CompilerParams(use_tc_tiling_on_sc=False),
    out_type=jax.ShapeDtypeStruct((NC, NROWS, D), jnp.float32),
    scratch_types=[
        pltpu.VMEM((BK,), jnp.int32),
        pltpu.VMEM((BK,), jnp.int32),
        pltpu.VMEM((BK, D), jnp.float32),
        pltpu.VMEM_SHARED((NROWS, D), jnp.float32),
        pltpu.SemaphoreType.DMA,
    ],
)
def _agg_kernel(g_hbm, src_hbm, dst_hbm, out_hbm, sidx_v, didx_v, rows_v, acc_sh, sem):
    cid = lax.axis_index("c")
    sid = lax.axis_index("s")
    wid = sid * NC + cid
    zv = jnp.zeros((16,), jnp.float32)
    for j in range(BK * D // 16):
        rows_v[j // (D // 16), pl.ds((j % (D // 16)) * 16, 16)] = zv
    for r in range(NROWS // NS // BK):
        pltpu.sync_copy(rows_v, acc_sh.at[pl.ds(sid * (NROWS // NS) + r * BK, BK)])
    plsc.subcore_barrier()

    def body(j, carry):
        base = wid * T_AGG + j * BK
        pltpu.sync_copy(src_hbm.at[pl.ds(base, BK)], sidx_v)
        pltpu.sync_copy(dst_hbm.at[pl.ds(base, BK)], didx_v)
        pltpu.async_copy(g_hbm.at[sidx_v], rows_v, sem).wait()
        pltpu.sync_copy(rows_v, acc_sh.at[didx_v], add=True)
        return carry

    lax.fori_loop(0, T_AGG // BK, body, 0)
    plsc.subcore_barrier()
    for r in range(NROWS // NS // BK):
        off = sid * (NROWS // NS) + r * BK
        pltpu.sync_copy(acc_sh.at[pl.ds(off, BK)], rows_v)
        pltpu.sync_copy(rows_v, out_hbm.at[cid, pl.ds(off, BK)])


@functools.partial(
    pl.kernel,
    mesh=_mesh,
    compiler_params=pltpu.CompilerParams(
        use_tc_tiling_on_sc=False, needs_layout_passes=False
    ),
    out_type=jax.ShapeDtypeStruct((E_DEC,), jnp.float32),
    scratch_types=[
        pltpu.VMEM((BK,), jnp.int32),
        pltpu.VMEM((BK,), jnp.int32),
        pltpu.VMEM((BK, D), jnp.float32),
        pltpu.VMEM((BK, D), jnp.float32),
        pltpu.VMEM((BK,), jnp.float32),
        pltpu.SemaphoreType.DMA,
    ],
)
def _decode_kernel(z_hbm, ia_hbm, ib_hbm, out_hbm, ia_v, ib_v, za_v, zb_v, o_v, sem):
    cid = lax.axis_index("c")
    sid = lax.axis_index("s")
    wid = sid * NC + cid

    def body(j, carry):
        base = wid * T_DEC + j * BK
        pltpu.sync_copy(ia_hbm.at[pl.ds(base, BK)], ia_v)
        pltpu.sync_copy(ib_hbm.at[pl.ds(base, BK)], ib_v)
        pltpu.async_copy(z_hbm.at[ia_v], za_v, sem).wait()
        pltpu.async_copy(z_hbm.at[ib_v], zb_v, sem).wait()
        lane = lax.iota(jnp.int32, 16)
        for g in range(BK // 16):
            res = jnp.zeros((16,), jnp.float32)
            for k in range(16):
                i = g * 16 + k
                acc = za_v[i, pl.ds(0, 16)] * zb_v[i, pl.ds(0, 16)]
                for q in range(1, D // 16):
                    acc = acc + za_v[i, pl.ds(q * 16, 16)] * zb_v[i, pl.ds(q * 16, 16)]
                res = jnp.where(lane == k, jnp.sum(acc), res)
            o_v[pl.ds(g * 16, 16)] = res
        pltpu.sync_copy(o_v, out_hbm.at[pl.ds(base, BK)])
        return carry

    lax.fori_loop(0, T_DEC // BK, body, 0)


def _enc_body(x_ref, degp_ref, W1r, b1r, W2r, b2r, W3r, b3r, Wc1r, g1_ref, dinv_ref):
    h = jnp.tanh(jnp.dot(x_ref[...], W1r[...], preferred_element_type=jnp.float32) + b1r[...])
    h = jnp.tanh(jnp.dot(h, W2r[...], preferred_element_type=jnp.float32) + b2r[...])
    h = jnp.tanh(jnp.dot(h, W3r[...], preferred_element_type=jnp.float32) + b3r[...])
    deg = 1.0 + degp_ref[0, :] + degp_ref[1, :]
    dinv = lax.rsqrt(deg)
    dinv_ref[...] = dinv
    g1_ref[...] = jnp.dot(h, Wc1r[...], preferred_element_type=jnp.float32) * dinv[:, None]


def _encoder(x_p, degp, W1, b1, W2, b2, W3, b3, Wc1):
    full = lambda a: pl.BlockSpec(a.shape, lambda i: (0,) * a.ndim)
    return pl.pallas_call(
        _enc_body,
        grid=(NROWS // RB,),
        in_specs=[
            pl.BlockSpec((RB, DF), lambda i: (i, 0)),
            pl.BlockSpec((NC, RB), lambda i: (0, i)),
            full(W1), full(b1), full(W2), full(b2), full(W3), full(b3), full(Wc1),
        ],
        out_specs=[
            pl.BlockSpec((RB, D), lambda i: (i, 0)),
            pl.BlockSpec((RB,), lambda i: (i,)),
        ],
        out_shape=[
            jax.ShapeDtypeStruct((NROWS, D), jnp.float32),
            jax.ShapeDtypeStruct((NROWS,), jnp.float32),
        ],
    )(x_p, degp, W1, b1, W2, b2, W3, b3, Wc1)


def _mid_body(acc_ref, g1_ref, dinv_ref, Wc2r, bc1r, g2_ref):
    a = acc_ref[0] + acc_ref[1] + g1_ref[...]
    dinv = dinv_ref[...]
    out1 = jax.nn.relu(bc1r[...] + dinv[:, None] * a)
    g2_ref[...] = jnp.dot(out1, Wc2r[...], preferred_element_type=jnp.float32) * dinv[:, None]


def _mid(acc1, g1, dinv, Wc2, bc1):
    full = lambda a: pl.BlockSpec(a.shape, lambda i: (0,) * a.ndim)
    return pl.pallas_call(
        _mid_body,
        grid=(NROWS // RB,),
        in_specs=[
            pl.BlockSpec((NC, RB, D), lambda i: (0, i, 0)),
            pl.BlockSpec((RB, D), lambda i: (i, 0)),
            pl.BlockSpec((RB,), lambda i: (i,)),
            full(Wc2), full(bc1),
        ],
        out_specs=pl.BlockSpec((RB, D), lambda i: (i, 0)),
        out_shape=jax.ShapeDtypeStruct((NROWS, D), jnp.float32),
    )(acc1, g1, dinv, Wc2, bc1)


def _fin_body(acc_ref, g2_ref, dinv_ref, W4r, bc2r, b4r, z_ref):
    a = acc_ref[0] + acc_ref[1] + g2_ref[...]
    out2 = bc2r[...] + dinv_ref[...][:, None] * a
    z_ref[...] = jnp.dot(out2, W4r[...], preferred_element_type=jnp.float32) + b4r[...]


def _fin(acc2, g2, dinv, W4, bc2, b4):
    full = lambda a: pl.BlockSpec(a.shape, lambda i: (0,) * a.ndim)
    return pl.pallas_call(
        _fin_body,
        grid=(NROWS // RB,),
        in_specs=[
            pl.BlockSpec((NC, RB, D), lambda i: (0, i, 0)),
            pl.BlockSpec((RB, D), lambda i: (i, 0)),
            pl.BlockSpec((RB,), lambda i: (i,)),
            full(W4), full(bc2), full(b4),
        ],
        out_specs=pl.BlockSpec((RB, D), lambda i: (i, 0)),
        out_shape=jax.ShapeDtypeStruct((NROWS, D), jnp.float32),
    )(acc2, g2, dinv, W4, bc2, b4)


@jax.jit
def kernel(x, train_pos_edge_index, pos_edge_index, neg_edge_index,
           W1, b1, W2, b2, W3, b3, Wc1, bc1, Wc2, bc2, W4, b4):
    i32 = jnp.int32
    src = train_pos_edge_index[0].astype(i32)
    dst = train_pos_edge_index[1].astype(i32)
    pad = jnp.full((E_AGG - E,), PADI, i32)
    src_p = jnp.concatenate([src, pad])
    dst_p = jnp.concatenate([dst, pad])

    x_p = jnp.pad(x, ((0, NROWS - N), (0, 0)))

    degp = _deg_kernel(dst_p)
    g1, dinv = _encoder(x_p, degp, W1, b1, W2, b2, W3, b3, Wc1)
    acc1 = _agg_kernel(g1, src_p, dst_p)
    g2 = _mid(acc1, g1, dinv, Wc2, bc1)
    acc2 = _agg_kernel(g2, src_p, dst_p)
    z = _fin(acc2, g2, dinv, W4, bc2, b4)

    dpad = jnp.zeros((E_DEC - E2,), i32)
    ia = jnp.concatenate([pos_edge_index[0].astype(i32), neg_edge_index[0].astype(i32), dpad])
    ib = jnp.concatenate([pos_edge_index[1].astype(i32), neg_edge_index[1].astype(i32), dpad])
    logits = _decode_kernel(z, ia, ib)
    return logits[:E2]


# speedup vs baseline: 11.0801x; 1.1936x over previous
import functools

import jax
import jax.numpy as jnp
from jax import lax
from jax.experimental import pallas as pl
from jax.experimental.pallas import tpu as pltpu
from jax.experimental.pallas import tpu_sc as plsc

N = 10000
E = 320000
E2 = 2 * E
DF = 128
D = 64

NROWS = 10240
RB = 1024
PADI = 10008

NC = 2
NS = 16
NW = NC * NS
BK = 128

T_AGG = 10240
E_AGG = NW * T_AGG
T_DEC = 20224
E_DEC = NW * T_DEC

_mesh = plsc.VectorSubcoreMesh(core_axis_name="c", subcore_axis_name="s")


def _worker_id():
    return lax.axis_index("s") * NC + lax.axis_index("c")


def _fill(ref, nwords, value, dtype):
    v = jnp.full((16,), value, dtype)
    for j in range(nwords // 16):
        ref[pl.ds(j * 16, 16)] = v


@functools.partial(
    pl.kernel,
    mesh=_mesh,
    out_type=jax.ShapeDtypeStruct((NC, NROWS), jnp.float32),
    scratch_types=[
        pltpu.VMEM((BK,), jnp.int32),
        pltpu.VMEM((BK,), jnp.float32),
        pltpu.VMEM_SHARED((NROWS,), jnp.float32),
    ],
)
def _deg_kernel(dst_hbm, out_hbm, idx_v, val_v, acc_sh):
    cid = lax.axis_index("c")
    sid = lax.axis_index("s")
    wid = sid * NC + cid
    _fill(val_v, BK, 0.0, jnp.float32)
    for r in range(NROWS // NS // BK):
        pltpu.sync_copy(val_v, acc_sh.at[pl.ds(sid * (NROWS // NS) + r * BK, BK)])
    plsc.subcore_barrier()
    _fill(val_v, BK, 1.0, jnp.float32)

    def body(j, carry):
        base = wid * T_AGG + j * BK
        pltpu.sync_copy(dst_hbm.at[pl.ds(base, BK)], idx_v)
        pltpu.sync_copy(val_v, acc_sh.at[idx_v], add=True)
        return carry

    lax.fori_loop(0, T_AGG // BK, body, 0)
    plsc.subcore_barrier()
    for r in range(NROWS // NS // BK):
        off = sid * (NROWS // NS) + r * BK
        pltpu.sync_copy(acc_sh.at[pl.ds(off, BK)], val_v)
        pltpu.sync_copy(val_v, out_hbm.at[cid, pl.ds(off, BK)])


@functools.partial(
    pl.kernel,
    mesh=_mesh,
    compiler_params=pltpu.CompilerParams(use_tc_tiling_on_sc=False),
    out_type=jax.ShapeDtypeStruct((NC, NROWS, D), jnp.float32),
    scratch_types=[
        pltpu.VMEM((BK,), jnp.int32),
        pltpu.VMEM((BK,), jnp.int32),
        pltpu.VMEM((BK,), jnp.int32),
        pltpu.VMEM((BK,), jnp.int32),
        pltpu.VMEM((BK, D), jnp.float32),
        pltpu.VMEM((BK, D), jnp.float32),
        pltpu.VMEM_SHARED((NROWS, D), jnp.float32),
        pltpu.SemaphoreType.DMA,
        pltpu.SemaphoreType.DMA,
        pltpu.SemaphoreType.DMA,
        pltpu.SemaphoreType.DMA,
    ],
)
def _agg_kernel(g_hbm, src_hbm, dst_hbm, out_hbm,
                sidx0, sidx1, didx0, didx1, rows0, rows1,
                acc_sh, sg0, sg1, ss0, ss1):
    cid = lax.axis_index("c")
    sid = lax.axis_index("s")
    wid = sid * NC + cid
    sidx = (sidx0, sidx1)
    didx = (didx0, didx1)
    rows = (rows0, rows1)
    sg = (sg0, sg1)
    ss = (ss0, ss1)
    nb = T_AGG // BK

    zv = jnp.zeros((16,), jnp.float32)
    for j in range(BK * D // 16):
        rows0[j // (D // 16), pl.ds((j % (D // 16)) * 16, 16)] = zv
    for r in range(NROWS // NS // BK):
        pltpu.sync_copy(rows0, acc_sh.at[pl.ds(sid * (NROWS // NS) + r * BK, BK)])
    plsc.subcore_barrier()

    def load_batch(b, p):
        base = wid * T_AGG + b * BK
        pltpu.sync_copy(src_hbm.at[pl.ds(base, BK)], sidx[p])
        pltpu.sync_copy(dst_hbm.at[pl.ds(base, BK)], didx[p])
        pltpu.async_copy(g_hbm.at[sidx[p]], rows[p], sg[p])

    load_batch(0, 0)

    def body(j2, carry):
        b0 = j2 * 2
        for p in range(2):
            b = b0 + p
            q = 1 - p
            if p == 0:
                pl.when(j2 > 0)(
                    lambda: pltpu.make_async_copy(rows[q], acc_sh.at[didx[q]], ss[q]).wait()
                )
            else:
                pltpu.make_async_copy(rows[q], acc_sh.at[didx[q]], ss[q]).wait()
            if p == 0:
                load_batch(b + 1, q)
            else:
                pl.when(j2 < nb // 2 - 1)(lambda: load_batch(b + 1, q))
            pltpu.make_async_copy(g_hbm.at[sidx[p]], rows[p], sg[p]).wait()
            pltpu.async_copy(rows[p], acc_sh.at[didx[p]], ss[p], add=True)
        return carry

    lax.fori_loop(0, nb // 2, body, 0)
    pltpu.make_async_copy(rows[1], acc_sh.at[didx[1]], ss[1]).wait()
    plsc.subcore_barrier()
    for r in range(NROWS // NS // BK):
        off = sid * (NROWS // NS) + r * BK
        pltpu.sync_copy(acc_sh.at[pl.ds(off, BK)], rows0)
        pltpu.sync_copy(rows0, out_hbm.at[cid, pl.ds(off, BK)])


@functools.partial(
    pl.kernel,
    mesh=_mesh,
    compiler_params=pltpu.CompilerParams(
        use_tc_tiling_on_sc=False, needs_layout_passes=False
    ),
    out_type=jax.ShapeDtypeStruct((E_DEC,), jnp.float32),
    scratch_types=[
        pltpu.VMEM((BK,), jnp.int32),
        pltpu.VMEM((BK,), jnp.int32),
        pltpu.VMEM((BK,), jnp.int32),
        pltpu.VMEM((BK,), jnp.int32),
        pltpu.VMEM((BK, D), jnp.float32),
        pltpu.VMEM((BK, D), jnp.float32),
        pltpu.VMEM((BK, D), jnp.float32),
        pltpu.VMEM((BK, D), jnp.float32),
        pltpu.VMEM((BK,), jnp.float32),
        pltpu.VMEM((BK,), jnp.float32),
        pltpu.SemaphoreType.DMA,
        pltpu.SemaphoreType.DMA,
        pltpu.SemaphoreType.DMA,
        pltpu.SemaphoreType.DMA,
    ],
)
def _decode_kernel(z_hbm, ia_hbm, ib_hbm, out_hbm,
                   ia0, ia1, ib0, ib1, za0, za1, zb0, zb1, o0, o1,
                   sg0, sg1, so0, so1):
    cid = lax.axis_index("c")
    sid = lax.axis_index("s")
    wid = sid * NC + cid
    ia = (ia0, ia1)
    ib = (ib0, ib1)
    za = (za0, za1)
    zb = (zb0, zb1)
    o = (o0, o1)
    sg = (sg0, sg1)
    so = (so0, so1)
    nb = T_DEC // BK

    def load_batch(b, p):
        base = wid * T_DEC + b * BK
        pltpu.sync_copy(ia_hbm.at[pl.ds(base, BK)], ia[p])
        pltpu.sync_copy(ib_hbm.at[pl.ds(base, BK)], ib[p])
        pltpu.async_copy(z_hbm.at[ia[p]], za[p], sg[p])
        pltpu.async_copy(z_hbm.at[ib[p]], zb[p], sg[p])

    load_batch(0, 0)

    def body(j2, carry):
        b0 = j2 * 2
        lane = lax.iota(jnp.int32, 16)
        for p in range(2):
            b = b0 + p
            q = 1 - p
            if p == 0:
                load_batch(b + 1, q)
            else:
                pl.when(j2 < nb // 2 - 1)(lambda: load_batch(b + 1, q))
            pltpu.make_async_copy(z_hbm.at[ia[p]], za[p], sg[p]).wait()
            pltpu.make_async_copy(z_hbm.at[ib[p]], zb[p], sg[p]).wait()
            pl.when(j2 > 0)(
                lambda: pltpu.make_async_copy(
                    o[p], out_hbm.at[pl.ds(wid * T_DEC, BK)], so[p]
                ).wait()
            )
            for g in range(BK // 16):
                res = jnp.zeros((16,), jnp.float32)
                for k in range(16):
                    i = g * 16 + k
                    acc = za[p][i, pl.ds(0, 16)] * zb[p][i, pl.ds(0, 16)]
                    for u in range(1, D // 16):
                        acc = acc + za[p][i, pl.ds(u * 16, 16)] * zb[p][i, pl.ds(u * 16, 16)]
                    res = jnp.where(lane == k, jnp.sum(acc), res)
                o[p][pl.ds(g * 16, 16)] = res
            pltpu.async_copy(o[p], out_hbm.at[pl.ds(wid * T_DEC + b * BK, BK)], so[p])
        return carry

    lax.fori_loop(0, nb // 2, body, 0)
    pltpu.make_async_copy(o[0], out_hbm.at[pl.ds(wid * T_DEC, BK)], so[0]).wait()
    pltpu.make_async_copy(o[1], out_hbm.at[pl.ds(wid * T_DEC, BK)], so[1]).wait()


def _enc_body(x_ref, degp_ref, W1r, b1r, W2r, b2r, W3r, b3r, Wc1r, g1_ref, dinv_ref):
    h = jnp.tanh(jnp.dot(x_ref[...], W1r[...], preferred_element_type=jnp.float32) + b1r[...])
    h = jnp.tanh(jnp.dot(h, W2r[...], preferred_element_type=jnp.float32) + b2r[...])
    h = jnp.tanh(jnp.dot(h, W3r[...], preferred_element_type=jnp.float32) + b3r[...])
    deg = 1.0 + degp_ref[0, :] + degp_ref[1, :]
    dinv = lax.rsqrt(deg)
    dinv_ref[...] = dinv
    g1_ref[...] = jnp.dot(h, Wc1r[...], preferred_element_type=jnp.float32) * dinv[:, None]


def _encoder(x_p, degp, W1, b1, W2, b2, W3, b3, Wc1):
    full = lambda a: pl.BlockSpec(a.shape, lambda i: (0,) * a.ndim)
    return pl.pallas_call(
        _enc_body,
        grid=(NROWS // RB,),
        in_specs=[
            pl.BlockSpec((RB, DF), lambda i: (i, 0)),
            pl.BlockSpec((NC, RB), lambda i: (0, i)),
            full(W1), full(b1), full(W2), full(b2), full(W3), full(b3), full(Wc1),
        ],
        out_specs=[
            pl.BlockSpec((RB, D), lambda i: (i, 0)),
            pl.BlockSpec((RB,), lambda i: (i,)),
        ],
        out_shape=[
            jax.ShapeDtypeStruct((NROWS, D), jnp.float32),
            jax.ShapeDtypeStruct((NROWS,), jnp.float32),
        ],
    )(x_p, degp, W1, b1, W2, b2, W3, b3, Wc1)


def _mid_body(acc_ref, g1_ref, dinv_ref, Wc2r, bc1r, g2_ref):
    a = acc_ref[0] + acc_ref[1] + g1_ref[...]
    dinv = dinv_ref[...]
    out1 = jax.nn.relu(bc1r[...] + dinv[:, None] * a)
    g2_ref[...] = jnp.dot(out1, Wc2r[...], preferred_element_type=jnp.float32) * dinv[:, None]


def _mid(acc1, g1, dinv, Wc2, bc1):
    full = lambda a: pl.BlockSpec(a.shape, lambda i: (0,) * a.ndim)
    return pl.pallas_call(
        _mid_body,
        grid=(NROWS // RB,),
        in_specs=[
            pl.BlockSpec((NC, RB, D), lambda i: (0, i, 0)),
            pl.BlockSpec((RB, D), lambda i: (i, 0)),
            pl.BlockSpec((RB,), lambda i: (i,)),
            full(Wc2), full(bc1),
        ],
        out_specs=pl.BlockSpec((RB, D), lambda i: (i, 0)),
        out_shape=jax.ShapeDtypeStruct((NROWS, D), jnp.float32),
    )(acc1, g1, dinv, Wc2, bc1)


def _fin_body(acc_ref, g2_ref, dinv_ref, W4r, bc2r, b4r, z_ref):
    a = acc_ref[0] + acc_ref[1] + g2_ref[...]
    out2 = bc2r[...] + dinv_ref[...][:, None] * a
    z_ref[...] = jnp.dot(out2, W4r[...], preferred_element_type=jnp.float32) + b4r[...]


def _fin(acc2, g2, dinv, W4, bc2, b4):
    full = lambda a: pl.BlockSpec(a.shape, lambda i: (0,) * a.ndim)
    return pl.pallas_call(
        _fin_body,
        grid=(NROWS // RB,),
        in_specs=[
            pl.BlockSpec((NC, RB, D), lambda i: (0, i, 0)),
            pl.BlockSpec((RB, D), lambda i: (i, 0)),
            pl.BlockSpec((RB,), lambda i: (i,)),
            full(W4), full(bc2), full(b4),
        ],
        out_specs=pl.BlockSpec((RB, D), lambda i: (i, 0)),
        out_shape=jax.ShapeDtypeStruct((NROWS, D), jnp.float32),
    )(acc2, g2, dinv, W4, bc2, b4)


@jax.jit
def kernel(x, train_pos_edge_index, pos_edge_index, neg_edge_index,
           W1, b1, W2, b2, W3, b3, Wc1, bc1, Wc2, bc2, W4, b4):
    i32 = jnp.int32
    src = train_pos_edge_index[0].astype(i32)
    dst = train_pos_edge_index[1].astype(i32)
    pad = jnp.full((E_AGG - E,), PADI, i32)
    src_p = jnp.concatenate([src, pad])
    dst_p = jnp.concatenate([dst, pad])

    x_p = jnp.pad(x, ((0, NROWS - N), (0, 0)))

    degp = _deg_kernel(dst_p)
    g1, dinv = _encoder(x_p, degp, W1, b1, W2, b2, W3, b3, Wc1)
    acc1 = _agg_kernel(g1, src_p, dst_p)
    g2 = _mid(acc1, g1, dinv, Wc2, bc1)
    acc2 = _agg_kernel(g2, src_p, dst_p)
    z = _fin(acc2, g2, dinv, W4, bc2, b4)

    dpad = jnp.zeros((E_DEC - E2,), i32)
    ia = jnp.concatenate([pos_edge_index[0].astype(i32), neg_edge_index[0].astype(i32), dpad])
    ib = jnp.concatenate([pos_edge_index[1].astype(i32), neg_edge_index[1].astype(i32), dpad])
    logits = _decode_kernel(z, ia, ib)
    return logits[:E2]
